# baseline (speedup 1.0000x reference)
import functools

import jax
import jax.numpy as jnp
from jax import lax
from jax.experimental import pallas as pl
from jax.experimental.pallas import tpu as pltpu
from jax.experimental.pallas import tpu_sc as plsc

N_NUC = 10000
N_CENTER = 320000
MAX_CHARGE = 100
FEAT = 128

NC, NS = 2, 16
NW = NC * NS
PER_W = N_CENTER // NW
CHUNK = 200
NCHUNK = PER_W // CHUNK
NB = 4
NRING = (NCHUNK + NB - 1) // NB

_mesh = plsc.VectorSubcoreMesh(core_axis_name="c", subcore_axis_name="s")


@functools.partial(
    pl.kernel,
    out_type=jax.ShapeDtypeStruct((N_CENTER, FEAT), jnp.float32),
    mesh=_mesh,
    scratch_types=(
        [pltpu.VMEM_SHARED((N_NUC,), jnp.int32)]
        + [pltpu.VMEM_SHARED((MAX_CHARGE, FEAT), jnp.float32)]
        + [pltpu.VMEM((CHUNK,), jnp.int32) for _ in range(NB)]
        + [pltpu.VMEM((CHUNK,), jnp.int32) for _ in range(NB)]
        + [pltpu.VMEM((CHUNK, FEAT), jnp.float32) for _ in range(NB)]
        + [pltpu.SemaphoreType.DMA for _ in range(4 * NB)]
    ),
)
def _two_level_gather(table_hbm, charges_hbm, cidx_hbm, out_hbm, *refs):
    fc_s = refs[0]
    tab_s = refs[1]
    cidx_v = refs[2:2 + NB]
    chg_v = refs[2 + NB:2 + 2 * NB]
    rows_v = refs[2 + 2 * NB:2 + 3 * NB]
    sems = refs[2 + 3 * NB:]
    sem_i = sems[0:NB]
    sem_c = sems[NB:2 * NB]
    sem_r = sems[2 * NB:3 * NB]
    sem_s = sems[3 * NB:4 * NB]

    wid = lax.axis_index("s") * NC + lax.axis_index("c")
    base = wid * PER_W

    pltpu.async_copy(
        cidx_hbm.at[pl.ds(base, CHUNK)], cidx_v[0], sem_i[0])
    pltpu.async_copy(
        cidx_hbm.at[pl.ds(base + CHUNK, CHUNK)], cidx_v[1], sem_i[1])

    @pl.when(lax.axis_index("s") == 0)
    def _():
        pltpu.sync_copy(charges_hbm, fc_s)

    @pl.when(lax.axis_index("s") == 1)
    def _():
        pltpu.sync_copy(table_hbm, tab_s)
    plsc.subcore_barrier()

    def fire_cidx(c, b):
        pltpu.async_copy(
            cidx_hbm.at[pl.ds(base + c * CHUNK, CHUNK)], cidx_v[b], sem_i[b])

    def wait_cidx(b):
        pltpu.make_async_copy(
            cidx_hbm.at[pl.ds(0, CHUNK)], cidx_v[b], sem_i[b]).wait()

    def fire_chg(b):
        pltpu.async_copy(fc_s.at[cidx_v[b]], chg_v[b], sem_c[b])

    def wait_chg(b):
        pltpu.make_async_copy(fc_s.at[cidx_v[b]], chg_v[b], sem_c[b]).wait()

    H = 104

    def fire_rows(b):
        pltpu.async_copy(
            tab_s.at[chg_v[b].at[pl.ds(0, H)]],
            rows_v[b].at[pl.ds(0, H)], sem_r[b])
        pltpu.async_copy(
            tab_s.at[chg_v[b].at[pl.ds(H, CHUNK - H)]],
            rows_v[b].at[pl.ds(H, CHUNK - H)], sem_r[b])

    def wait_rows(b):
        pltpu.make_async_copy(
            tab_s.at[chg_v[b].at[pl.ds(0, H)]],
            rows_v[b].at[pl.ds(0, H)], sem_r[b]).wait()
        pltpu.make_async_copy(
            tab_s.at[chg_v[b].at[pl.ds(H, CHUNK - H)]],
            rows_v[b].at[pl.ds(H, CHUNK - H)], sem_r[b]).wait()

    def fire_store(c, b):
        pltpu.async_copy(
            rows_v[b], out_hbm.at[pl.ds(base + c * CHUNK, CHUNK)], sem_s[b])

    def wait_store(b):
        pltpu.make_async_copy(
            rows_v[b], out_hbm.at[pl.ds(0, CHUNK)], sem_s[b]).wait()

    wait_cidx(0)
    fire_chg(0)

    @pl.loop(0, NRING)
    def _(g):
        for b in range(NB):
            c = g * NB + b

            @pl.when(c < NCHUNK)
            def _():
                @pl.when(c >= NB)
                def _():
                    wait_store(b)
                wait_chg(b)
                fire_rows(b)

                @pl.when(c + 2 < NCHUNK)
                def _():
                    fire_cidx(c + 2, (b + 2) % NB)

                @pl.when(c >= 1)
                def _():
                    pb = (b - 1) % NB
                    wait_rows(pb)
                    fire_store(c - 1, pb)

                @pl.when(c + 1 < NCHUNK)
                def _():
                    nb_ = (b + 1) % NB
                    wait_cidx(nb_)
                    fire_chg(nb_)

    last_b = (NCHUNK - 1) % NB
    wait_rows(last_b)
    fire_store(NCHUNK - 1, last_b)
    for b in range(NB):
        wait_store(b)


def kernel(table, flat_charges, center_idx):
    return _two_level_gather(
        table,
        flat_charges.astype(jnp.int32),
        center_idx.astype(jnp.int32),
    )


# speedup vs baseline: 1.0002x; 1.0002x over previous
import functools

import jax
import jax.numpy as jnp
from jax import lax
from jax.experimental import pallas as pl
from jax.experimental.pallas import tpu as pltpu
from jax.experimental.pallas import tpu_sc as plsc

N_NUC = 10000
N_CENTER = 320000
MAX_CHARGE = 100
FEAT = 128

NC, NS = 2, 16
NW = NC * NS
PER_W = N_CENTER // NW
CHUNK = 200
NCHUNK = PER_W // CHUNK
NB = 4
NRING = (NCHUNK + NB - 1) // NB

_mesh = plsc.VectorSubcoreMesh(core_axis_name="c", subcore_axis_name="s")


@functools.partial(
    pl.kernel,
    out_type=jax.ShapeDtypeStruct((N_CENTER, FEAT), jnp.float32),
    mesh=_mesh,
    scratch_types=(
        [pltpu.VMEM_SHARED((N_NUC,), jnp.int32)]
        + [pltpu.VMEM_SHARED((MAX_CHARGE, FEAT), jnp.float32)]
        + [pltpu.VMEM((CHUNK,), jnp.int32) for _ in range(NB)]
        + [pltpu.VMEM((CHUNK,), jnp.int32) for _ in range(NB)]
        + [pltpu.VMEM((CHUNK, FEAT), jnp.float32) for _ in range(NB)]
        + [pltpu.SemaphoreType.DMA for _ in range(4 * NB)]
    ),
)
def _two_level_gather(table_hbm, charges_hbm, cidx_hbm, out_hbm, *refs):
    fc_s = refs[0]
    tab_s = refs[1]
    cidx_v = refs[2:2 + NB]
    chg_v = refs[2 + NB:2 + 2 * NB]
    rows_v = refs[2 + 2 * NB:2 + 3 * NB]
    sems = refs[2 + 3 * NB:]
    sem_i = sems[0:NB]
    sem_c = sems[NB:2 * NB]
    sem_r = sems[2 * NB:3 * NB]
    sem_s = sems[3 * NB:4 * NB]

    wid = lax.axis_index("s") * NC + lax.axis_index("c")
    base = wid * PER_W

    pltpu.async_copy(
        cidx_hbm.at[pl.ds(base, CHUNK)], cidx_v[0], sem_i[0])
    pltpu.async_copy(
        cidx_hbm.at[pl.ds(base + CHUNK, CHUNK)], cidx_v[1], sem_i[1])

    @pl.when(lax.axis_index("s") == 0)
    def _():
        pltpu.sync_copy(charges_hbm, fc_s)

    @pl.when(lax.axis_index("s") == 1)
    def _():
        pltpu.sync_copy(table_hbm, tab_s)
    plsc.subcore_barrier()

    def fire_cidx(c, b):
        pltpu.async_copy(
            cidx_hbm.at[pl.ds(base + c * CHUNK, CHUNK)], cidx_v[b], sem_i[b])

    def wait_cidx(b):
        pltpu.make_async_copy(
            cidx_hbm.at[pl.ds(0, CHUNK)], cidx_v[b], sem_i[b]).wait()

    def fire_chg(b):
        pltpu.async_copy(fc_s.at[cidx_v[b]], chg_v[b], sem_c[b])

    def wait_chg(b):
        pltpu.make_async_copy(fc_s.at[cidx_v[b]], chg_v[b], sem_c[b]).wait()

    def fire_rows(b):
        pltpu.async_copy(tab_s.at[chg_v[b]], rows_v[b], sem_r[b])

    def wait_rows(b):
        pltpu.make_async_copy(tab_s.at[chg_v[b]], rows_v[b], sem_r[b]).wait()

    def fire_store(c, b):
        pltpu.async_copy(
            rows_v[b], out_hbm.at[pl.ds(base + c * CHUNK, CHUNK)], sem_s[b])

    def wait_store(b):
        pltpu.make_async_copy(
            rows_v[b], out_hbm.at[pl.ds(0, CHUNK)], sem_s[b]).wait()

    wait_cidx(0)
    fire_chg(0)

    @pl.loop(0, NRING)
    def _(g):
        for b in range(NB):
            c = g * NB + b

            @pl.when(c < NCHUNK)
            def _():
                @pl.when(c >= NB)
                def _():
                    wait_store(b)
                wait_chg(b)
                fire_rows(b)

                @pl.when(c + 2 < NCHUNK)
                def _():
                    fire_cidx(c + 2, (b + 2) % NB)

                @pl.when(c >= 1)
                def _():
                    pb = (b - 1) % NB
                    wait_rows(pb)
                    fire_store(c - 1, pb)

                @pl.when(c + 1 < NCHUNK)
                def _():
                    nb_ = (b + 1) % NB
                    wait_cidx(nb_)
                    fire_chg(nb_)

    last_b = (NCHUNK - 1) % NB
    wait_rows(last_b)
    fire_store(NCHUNK - 1, last_b)
    for b in range(NB):
        wait_store(b)


def kernel(table, flat_charges, center_idx):
    return _two_level_gather(
        table,
        flat_charges.astype(jnp.int32),
        center_idx.astype(jnp.int32),
    )


# speedup vs baseline: 1.0198x; 1.0197x over previous
import functools

import jax
import jax.numpy as jnp
from jax import lax
from jax.experimental import pallas as pl
from jax.experimental.pallas import tpu as pltpu
from jax.experimental.pallas import tpu_sc as plsc

N_NUC = 10000
N_CENTER = 320000
MAX_CHARGE = 100
FEAT = 128

NC, NS = 2, 16
NW = NC * NS
PER_W = N_CENTER // NW
CHUNK = 200
NCHUNK = PER_W // CHUNK
NB = 4
NRING = (NCHUNK + NB - 1) // NB

_mesh = plsc.VectorSubcoreMesh(core_axis_name="c", subcore_axis_name="s")


@functools.partial(
    pl.kernel,
    out_type=jax.ShapeDtypeStruct((N_CENTER, FEAT), jnp.float32),
    mesh=_mesh,
    scratch_types=(
        [pltpu.VMEM_SHARED((N_NUC,), jnp.int32)]
        + [pltpu.VMEM_SHARED((MAX_CHARGE, FEAT), jnp.float32)]
        + [pltpu.VMEM((CHUNK,), jnp.int32) for _ in range(NB)]
        + [pltpu.VMEM((CHUNK,), jnp.int32) for _ in range(NB)]
        + [pltpu.VMEM((CHUNK, FEAT), jnp.float32) for _ in range(NB)]
        + [pltpu.SemaphoreType.DMA for _ in range(4 * NB)]
    ),
)
def _two_level_gather(table_hbm, charges_hbm, cidx_hbm, out_hbm, *refs):
    fc_s = refs[0]
    tab_s = refs[1]
    cidx_v = refs[2:2 + NB]
    chg_v = refs[2 + NB:2 + 2 * NB]
    rows_v = refs[2 + 2 * NB:2 + 3 * NB]
    sems = refs[2 + 3 * NB:]
    sem_i = sems[0:NB]
    sem_c = sems[NB:2 * NB]
    sem_r = sems[2 * NB:3 * NB]
    sem_s = sems[3 * NB:4 * NB]

    wid = lax.axis_index("s") * NC + lax.axis_index("c")
    base = wid * PER_W

    pltpu.async_copy(
        cidx_hbm.at[pl.ds(base, CHUNK)], cidx_v[0], sem_i[0])
    pltpu.async_copy(
        cidx_hbm.at[pl.ds(base + CHUNK, CHUNK)], cidx_v[1], sem_i[1])

    @pl.when(lax.axis_index("s") == 0)
    def _():
        pltpu.sync_copy(charges_hbm, fc_s)

    @pl.when(lax.axis_index("s") == 1)
    def _():
        pltpu.sync_copy(table_hbm, tab_s)
    plsc.subcore_barrier()

    def fire_cidx(c, b):
        pltpu.async_copy(
            cidx_hbm.at[pl.ds(base + c * CHUNK, CHUNK)], cidx_v[b], sem_i[b])

    def wait_cidx(b):
        pltpu.make_async_copy(
            cidx_hbm.at[pl.ds(0, CHUNK)], cidx_v[b], sem_i[b]).wait()

    def fire_chg(b):
        pltpu.async_copy(fc_s.at[cidx_v[b]], chg_v[b], sem_c[b])

    def wait_chg(b):
        pltpu.make_async_copy(fc_s.at[cidx_v[b]], chg_v[b], sem_c[b]).wait()

    def fire_rows(b):
        pltpu.async_copy(tab_s.at[chg_v[b]], rows_v[b], sem_r[b])

    def wait_rows(b):
        pltpu.make_async_copy(tab_s.at[chg_v[b]], rows_v[b], sem_r[b]).wait()

    def fire_store(c, b):
        pltpu.async_copy(
            rows_v[b], out_hbm.at[pl.ds(base + c * CHUNK, CHUNK)], sem_s[b])

    def wait_store(b):
        pltpu.make_async_copy(
            rows_v[b], out_hbm.at[pl.ds(0, CHUNK)], sem_s[b]).wait()

    wait_cidx(0)
    fire_chg(0)

    @pl.loop(0, NRING)
    def _(g):
        for b in range(NB):
            c = g * NB + b

            @pl.when(c < NCHUNK)
            def _():
                @pl.when(c >= NB)
                def _():
                    wait_store(b)
                wait_chg(b)
                fire_rows(b)

                @pl.when(c + 2 < NCHUNK)
                def _():
                    fire_cidx(c + 2, (b + 2) % NB)

                @pl.when(c + 1 < NCHUNK)
                def _():
                    nb_ = (b + 1) % NB
                    wait_cidx(nb_)
                    fire_chg(nb_)

                @pl.when(c >= 1)
                def _():
                    pb = (b - 1) % NB
                    wait_rows(pb)
                    fire_store(c - 1, pb)

    last_b = (NCHUNK - 1) % NB
    wait_rows(last_b)
    fire_store(NCHUNK - 1, last_b)
    for b in range(NB):
        wait_store(b)


def kernel(table, flat_charges, center_idx):
    return _two_level_gather(
        table,
        flat_charges.astype(jnp.int32),
        center_idx.astype(jnp.int32),
    )
